# baseline (speedup 1.0000x reference)
import functools

import jax
import jax.numpy as jnp
from jax import lax
from jax.experimental import pallas as pl
from jax.experimental.pallas import tpu as pltpu
from jax.experimental.pallas import tpu_sc as plsc

_D = 64
_NW = 32
_CHUNK = 800
_NBUF = 2


@functools.lru_cache(maxsize=None)
def _build(B):
    b_per_w = B // _NW
    n_chunks = b_per_w // _CHUNK
    mesh = plsc.VectorSubcoreMesh(core_axis_name="c", subcore_axis_name="s")

    @functools.partial(
        pl.kernel,
        mesh=mesh,
        compiler_params=pltpu.CompilerParams(use_tc_tiling_on_sc=False),
        out_type=jax.ShapeDtypeStruct((B, _D), jnp.float32),
        scratch_types=[
            pltpu.VMEM((b_per_w,), jnp.int32),
            pltpu.VMEM((_NBUF, _CHUNK, _D), jnp.float32),
            pltpu.SemaphoreType.DMA,
            pltpu.SemaphoreType.DMA,
            pltpu.SemaphoreType.DMA,
            pltpu.SemaphoreType.DMA,
        ],
    )
    def emb(idx_hbm, table_hbm, out_hbm, idx_v, rows_v, g0, g1, o0, o1):
        gsem = (g0, g1)
        osem = (o0, o1)
        wid = lax.axis_index("s") * 2 + lax.axis_index("c")
        base = wid * b_per_w
        pltpu.sync_copy(idx_hbm.at[pl.ds(base, b_per_w)], idx_v)

        gathers = [None] * _NBUF
        outs = [None] * _NBUF
        for i in range(n_chunks):
            b = i % _NBUF
            if outs[b] is not None:
                outs[b].wait()
            gathers[b] = pltpu.async_copy(
                table_hbm.at[idx_v.at[pl.ds(i * _CHUNK, _CHUNK)]],
                rows_v.at[b], gsem[b])
            if i > 0:
                pb = (i - 1) % _NBUF
                gathers[pb].wait()
                outs[pb] = pltpu.async_copy(
                    rows_v.at[pb],
                    out_hbm.at[pl.ds(base + (i - 1) * _CHUNK, _CHUNK)],
                    osem[pb])
        last = n_chunks - 1
        lb = last % _NBUF
        gathers[lb].wait()
        outs[lb] = pltpu.async_copy(
            rows_v.at[lb],
            out_hbm.at[pl.ds(base + last * _CHUNK, _CHUNK)],
            osem[lb])
        for b in range(_NBUF):
            if outs[b] is not None:
                outs[b].wait()

    return emb


def kernel(input_ids, table):
    bt, h = input_ids.shape
    flat = input_ids.reshape(bt * h).astype(jnp.int32)
    out = _build(bt * h)(flat, table)
    return out.reshape(bt, h, _D)


# speedup vs baseline: 1.0056x; 1.0056x over previous
import functools

import jax
import jax.numpy as jnp
from jax import lax
from jax.experimental import pallas as pl
from jax.experimental.pallas import tpu as pltpu
from jax.experimental.pallas import tpu_sc as plsc

_D = 64
_DP = 128
_NW = 32
_CHUNK = 400
_NBUF = 2


@functools.lru_cache(maxsize=None)
def _build(B):
    b_per_w = B // _NW
    n_chunks = b_per_w // _CHUNK
    mesh = plsc.VectorSubcoreMesh(core_axis_name="c", subcore_axis_name="s")

    @functools.partial(
        pl.kernel,
        mesh=mesh,
        compiler_params=pltpu.CompilerParams(use_tc_tiling_on_sc=True),
        out_type=jax.ShapeDtypeStruct((B, _DP), jnp.float32),
        scratch_types=[
            pltpu.VMEM((b_per_w,), jnp.int32),
            pltpu.VMEM((_NBUF, _CHUNK, _DP), jnp.float32),
            pltpu.SemaphoreType.DMA,
            pltpu.SemaphoreType.DMA,
            pltpu.SemaphoreType.DMA,
            pltpu.SemaphoreType.DMA,
        ],
    )
    def emb(idx_hbm, table_hbm, out_hbm, idx_v, rows_v, g0, g1, o0, o1):
        gsem = (g0, g1)
        osem = (o0, o1)
        wid = lax.axis_index("s") * 2 + lax.axis_index("c")
        base = wid * b_per_w
        pltpu.sync_copy(idx_hbm.at[pl.ds(base, b_per_w)], idx_v)

        gathers = [None] * _NBUF
        outs = [None] * _NBUF
        for i in range(n_chunks):
            b = i % _NBUF
            if outs[b] is not None:
                outs[b].wait()
            gathers[b] = pltpu.async_copy(
                table_hbm.at[idx_v.at[pl.ds(i * _CHUNK, _CHUNK)]],
                rows_v.at[b], gsem[b])
            if i > 0:
                pb = (i - 1) % _NBUF
                gathers[pb].wait()
                outs[pb] = pltpu.async_copy(
                    rows_v.at[pb],
                    out_hbm.at[pl.ds(base + (i - 1) * _CHUNK, _CHUNK)],
                    osem[pb])
        last = n_chunks - 1
        lb = last % _NBUF
        gathers[lb].wait()
        outs[lb] = pltpu.async_copy(
            rows_v.at[lb],
            out_hbm.at[pl.ds(base + last * _CHUNK, _CHUNK)],
            osem[lb])
        for b in range(_NBUF):
            if outs[b] is not None:
                outs[b].wait()

    return emb


def kernel(input_ids, table):
    bt, h = input_ids.shape
    flat = input_ids.reshape(bt * h).astype(jnp.int32)
    tpad = jnp.pad(table, ((0, 0), (0, _DP - _D)))
    out = _build(bt * h)(flat, tpad)
    return out[:, :_D].reshape(bt, h, _D)
